# baseline (speedup 1.0000x reference)
import jax
import jax.numpy as jnp
from jax.experimental import pallas as pl
from jax.experimental.pallas import tpu as pltpu


_BK = 512


def _nn_body(q_ref, kt_ref, dist_ref, idx_ref, m_ref, mi_ref):
    i = pl.program_id(0)
    nblk = pl.num_programs(0)

    qx = q_ref[:, 0:1]
    qy = q_ref[:, 1:2]
    qz = q_ref[:, 2:3]
    x2 = (qx * qx + qy * qy) + qz * qz

    kx = kt_ref[0:1, :]
    ky = kt_ref[1:2, :]
    kz = kt_ref[2:3, :]
    y2 = (kx * kx + ky * ky) + kz * kz

    qb = q_ref[...].astype(jnp.bfloat16) * jnp.bfloat16(-2)
    kb = kt_ref[...].astype(jnp.bfloat16)
    t = jax.lax.dot_general(
        qb, kb, (((1,), (0,)), ((), ())),
        preferred_element_type=jnp.float32)
    d2 = (x2 + y2) + t
    d = jnp.sqrt(jnp.maximum(d2, 1e-12))

    v = jnp.min(d, axis=1, keepdims=True)
    kidx = jax.lax.broadcasted_iota(jnp.int32, (1, _BK), 1) + i * _BK
    cand = jnp.where(d == v, kidx, jnp.int32(2**31 - 1))
    a = jnp.min(cand, axis=1, keepdims=True)

    @pl.when(i == 0)
    def _init():
        m_ref[...] = v
        mi_ref[...] = a

    @pl.when(i > 0)
    def _update():
        m = m_ref[...]
        mask = v < m
        m_ref[...] = jnp.where(mask, v, m)
        mi_ref[...] = jnp.where(mask, a, mi_ref[...])

    @pl.when(i == nblk - 1)
    def _finalize():
        dist_ref[...] = m_ref[...]
        idx_ref[...] = mi_ref[...]


def kernel(queries, keys, k):
    q, dim = queries.shape
    nk = keys.shape[0]
    pdim = 8
    queries_p = jnp.pad(queries, ((0, 0), (0, pdim - dim)))
    keys_t = jnp.pad(keys.T, ((0, pdim - dim), (0, 0)))
    nblk = nk // _BK

    dist, idx = pl.pallas_call(
        _nn_body,
        grid=(nblk,),
        in_specs=[
            pl.BlockSpec((q, pdim), lambda i: (0, 0)),
            pl.BlockSpec((pdim, _BK), lambda i: (0, i)),
        ],
        out_specs=[
            pl.BlockSpec((q, 1), lambda i: (0, 0)),
            pl.BlockSpec((q, 1), lambda i: (0, 0)),
        ],
        out_shape=[
            jax.ShapeDtypeStruct((q, 1), jnp.float32),
            jax.ShapeDtypeStruct((q, 1), jnp.int32),
        ],
        scratch_shapes=[
            pltpu.VMEM((q, 1), jnp.float32),
            pltpu.VMEM((q, 1), jnp.int32),
        ],
    )(queries_p, keys_t)

    idx = idx + (jnp.asarray(k, dtype=idx.dtype) - 1)
    return (dist, idx, idx[:, 0])


# speedup vs baseline: 1.1096x; 1.1096x over previous
import jax
import jax.numpy as jnp
from jax import lax
from jax.experimental import pallas as pl
from jax.experimental.pallas import tpu as pltpu
from jax.experimental.pallas import tpu_sc as plsc


_BK = 512
_NK = 65536
_NQ = 1024
_NBLK = _NK // _BK
_NWORK = 32
_QPW = _NQ // _NWORK
_BIG = 2**30


def _phase1_body(q_ref, kt_ref, dist_ref, hi_ref, mb_ref, y2_ref, x2_ref,
                 m_ref):
    i = pl.program_id(0)
    nblk = pl.num_programs(0)

    qx = q_ref[:, 0:1]
    qy = q_ref[:, 1:2]
    qz = q_ref[:, 2:3]
    x2 = (qx * qx + qy * qy) + qz * qz

    kx = kt_ref[0:1, :]
    ky = kt_ref[1:2, :]
    kz = kt_ref[2:3, :]
    y2 = (kx * kx + ky * ky) + kz * kz

    qb = q_ref[...].astype(jnp.bfloat16) * jnp.bfloat16(-2)
    kb = kt_ref[...].astype(jnp.bfloat16)
    t = jax.lax.dot_general(
        qb, kb, (((1,), (0,)), ((), ())),
        preferred_element_type=jnp.float32)
    d2 = (x2 + y2) + t

    bm = jnp.min(d2, axis=1, keepdims=True)
    mb_ref[...] = bm[None]
    y2_ref[...] = y2[None]
    x2_ref[...] = x2

    @pl.when(i == 0)
    def _init():
        m_ref[...] = bm

    @pl.when(i > 0)
    def _update():
        m_ref[...] = jnp.minimum(m_ref[...], bm)

    @pl.when(i == nblk - 1)
    def _finalize():
        m = m_ref[...]
        mc = jnp.maximum(m, 1e-12)
        s = jnp.sqrt(mc)
        dist_ref[...] = s
        p = s * s
        pi = lax.bitcast_convert_type(p, jnp.int32)
        hi = mc
        for u in range(-4, 5):
            xu = lax.bitcast_convert_type(pi + u, jnp.float32)
            ok = jnp.sqrt(xu) <= s
            hi = jnp.maximum(hi, jnp.where(ok, xu, mc))
        hi_ref[...] = hi


def _phase2_body(hi_hbm, mb_hbm, qtm_hbm, kc_hbm, x2_hbm, y2_hbm, out_hbm,
                 hi_v, mb_v, qtm_v, x2_v, kc_v, y2_v, res_v):
    c = lax.axis_index("c")
    s_ = lax.axis_index("s")
    wid = s_ * 2 + c
    base = wid * _QPW

    pltpu.sync_copy(hi_hbm.at[pl.ds(base, _QPW)], hi_v.at[pl.ds(0, _QPW)])
    pltpu.sync_copy(mb_hbm.at[pl.ds(base, _QPW)], mb_v)
    pltpu.sync_copy(qtm_hbm.at[:, pl.ds(base, _QPW)],
                    qtm_v.at[:, pl.ds(0, _QPW)])
    pltpu.sync_copy(x2_hbm.at[pl.ds(base, _QPW)], x2_v.at[pl.ds(0, _QPW)])

    iota = lax.iota(jnp.int32, 16)
    iotaf = iota.astype(jnp.float32)
    bigf = jnp.full((16,), 16777216.0, jnp.float32)
    clipv = jnp.full((16,), 1e-12, jnp.float32)

    def qbody(i, carry):
        r0, r1 = carry
        hi_q = hi_v[pl.ds(i, 16)][0]
        bhi = jnp.full((16,), hi_q, jnp.float32)

        bmin = bigf
        for cch in range(_NBLK // 16):
            mbv = mb_v[i, pl.ds(cch * 16, 16)]
            mbc = jnp.maximum(mbv, clipv)
            cand = jnp.where(mbc <= bhi, cch * 16 + iotaf, bigf)
            bmin = jnp.minimum(bmin, cand)
        bstar = jnp.min(bmin).astype(jnp.int32)

        off = pl.multiple_of(bstar * _BK, _BK)
        pltpu.sync_copy(kc_hbm.at[:, pl.ds(off, _BK)], kc_v)
        pltpu.sync_copy(y2_hbm.at[bstar], y2_v)

        def bfr(x):
            u = lax.bitcast_convert_type(x, jnp.int32)
            r = (u + 0x7FFF + ((u >> 16) & 1)) & (-65536)
            return lax.bitcast_convert_type(r, jnp.float32)

        bqxm = bfr(jnp.full((16,), qtm_v[0, pl.ds(i, 16)][0], jnp.float32)) * -2.0
        bqym = bfr(jnp.full((16,), qtm_v[1, pl.ds(i, 16)][0], jnp.float32)) * -2.0
        bqzm = bfr(jnp.full((16,), qtm_v[2, pl.ds(i, 16)][0], jnp.float32)) * -2.0
        bx2 = jnp.full((16,), x2_v[pl.ds(i, 16)][0], jnp.float32)
        amask = bx2 == bx2

        rmin = bigf
        for j in range(_BK // 16):
            sl = pl.ds(j * 16, 16)
            kxv = kc_v[0, sl]
            kyv = kc_v[1, sl]
            kzv = kc_v[2, sl]
            y2 = y2_v[sl]
            def armor(x):
                return jnp.where(amask, x, bigf)

            p0 = bqxm * bfr(kxv)
            p1 = bqym * bfr(kyv)
            p2 = bqzm * bfr(kzv)
            s1 = armor(p0 + p1)
            bb1 = armor(s1 - p0)
            e1 = armor(p0 - (s1 - bb1)) + armor(p1 - bb1)
            s2 = armor(s1 + p2)
            bb2 = armor(s2 - s1)
            e2 = armor(s1 - (s2 - bb2)) + armor(p2 - bb2)
            t = s2 + armor(e1 + e2)
            d2 = (bx2 + y2) + t
            d2c = jnp.maximum(d2, clipv)
            cand = jnp.where(d2c <= bhi, j * 16 + iotaf, bigf)
            rmin = jnp.minimum(rmin, cand)
        fi = jnp.min(rmin).astype(jnp.int32)
        res = bstar * _BK + fi

        lane = jnp.where(iota == (i & 15), res, 0)
        grp = i >> 4
        r0 = jnp.where(jnp.full((16,), grp == 0), r0 | lane, r0)
        r1 = jnp.where(jnp.full((16,), grp == 1), r1 | lane, r1)
        return (r0, r1)

    z = jnp.zeros((16,), jnp.int32)
    r0, r1 = lax.fori_loop(0, _QPW, qbody, (z, z))
    res_v[pl.ds(0, 16)] = r0
    res_v[pl.ds(16, 16)] = r1
    pltpu.sync_copy(res_v, out_hbm.at[pl.ds(base, _QPW)])


def kernel(queries, keys, k):
    q, dim = queries.shape
    pdim = 8
    queries_p = jnp.pad(queries, ((0, 0), (0, pdim - dim)))
    keys_t = jnp.pad(keys.T, ((0, pdim - dim), (0, 0)))

    dist, hi, mb, y2a, x2o = pl.pallas_call(
        _phase1_body,
        grid=(_NBLK,),
        in_specs=[
            pl.BlockSpec((q, pdim), lambda i: (0, 0)),
            pl.BlockSpec((pdim, _BK), lambda i: (0, i)),
        ],
        out_specs=[
            pl.BlockSpec((q, 1), lambda i: (0, 0)),
            pl.BlockSpec((q, 1), lambda i: (0, 0)),
            pl.BlockSpec((1, q, 1), lambda i: (i, 0, 0)),
            pl.BlockSpec((1, 1, _BK), lambda i: (i, 0, 0)),
            pl.BlockSpec((q, 1), lambda i: (0, 0)),
        ],
        out_shape=[
            jax.ShapeDtypeStruct((q, 1), jnp.float32),
            jax.ShapeDtypeStruct((q, 1), jnp.float32),
            jax.ShapeDtypeStruct((_NBLK, q, 1), jnp.float32),
            jax.ShapeDtypeStruct((_NBLK, 1, _BK), jnp.float32),
            jax.ShapeDtypeStruct((q, 1), jnp.float32),
        ],
        scratch_shapes=[
            pltpu.VMEM((q, 1), jnp.float32),
        ],
    )(queries_p, keys_t)

    kcomb = keys.T
    qtm = queries.T

    mesh = plsc.VectorSubcoreMesh(core_axis_name="c", subcore_axis_name="s")
    phase2 = pl.kernel(
        _phase2_body,
        mesh=mesh,
        out_type=jax.ShapeDtypeStruct((q,), jnp.int32),
        compiler_params=pltpu.CompilerParams(needs_layout_passes=False),
        scratch_types=[
            pltpu.VMEM((_QPW + 16,), jnp.float32),
            pltpu.VMEM((_QPW, _NBLK), jnp.float32),
            pltpu.VMEM((3, _QPW + 16), jnp.float32),
            pltpu.VMEM((_QPW + 16,), jnp.float32),
            pltpu.VMEM((3, _BK), jnp.float32),
            pltpu.VMEM((_BK,), jnp.float32),
            pltpu.VMEM((_QPW,), jnp.int32),
        ],
    )
    mb_t = mb.reshape(_NBLK, q).T
    idxs = phase2(hi[:, 0], mb_t, qtm, kcomb,
                  x2o[:, 0], y2a.reshape(_NBLK, _BK))

    idx = idxs[:, None] + (jnp.asarray(k, dtype=idxs.dtype) - 1)
    return (dist, idx, idx[:, 0])


# speedup vs baseline: 1.4091x; 1.2699x over previous
import jax
import jax.numpy as jnp
from jax import lax
from jax.experimental import pallas as pl
from jax.experimental.pallas import tpu as pltpu
from jax.experimental.pallas import tpu_sc as plsc


_BK = 1024
_NK = 65536
_NQ = 1024
_NBLK = _NK // _BK
_NWORK = 32
_QPW = _NQ // _NWORK
_BIG = 2**30


def _phase1_body(q_ref, kt_ref, dist_ref, hi_ref, mb_ref, y2_ref, x2_ref,
                 m_ref):
    i = pl.program_id(0)
    nblk = pl.num_programs(0)

    qx = q_ref[:, 0:1]
    qy = q_ref[:, 1:2]
    qz = q_ref[:, 2:3]
    x2 = (qx * qx + qy * qy) + qz * qz

    kx = kt_ref[0:1, :]
    ky = kt_ref[1:2, :]
    kz = kt_ref[2:3, :]
    y2 = (kx * kx + ky * ky) + kz * kz

    qb = q_ref[...].astype(jnp.bfloat16) * jnp.bfloat16(-2)
    kb = kt_ref[...].astype(jnp.bfloat16)
    t = jax.lax.dot_general(
        qb, kb, (((1,), (0,)), ((), ())),
        preferred_element_type=jnp.float32)
    d2 = (x2 + y2) + t

    bm = jnp.min(d2, axis=1, keepdims=True)
    mb_ref[...] = bm[None]
    y2_ref[...] = y2[None]
    x2_ref[...] = x2

    @pl.when(i == 0)
    def _init():
        m_ref[...] = bm

    @pl.when(i > 0)
    def _update():
        m_ref[...] = jnp.minimum(m_ref[...], bm)

    @pl.when(i == nblk - 1)
    def _finalize():
        m = m_ref[...]
        mc = jnp.maximum(m, 1e-12)
        s = jnp.sqrt(mc)
        dist_ref[...] = s
        p = s * s
        pi = lax.bitcast_convert_type(p, jnp.int32)
        hi = mc
        for u in range(-4, 5):
            xu = lax.bitcast_convert_type(pi + u, jnp.float32)
            ok = jnp.sqrt(xu) <= s
            hi = jnp.maximum(hi, jnp.where(ok, xu, mc))
        hi_ref[...] = hi


def _phase2_body(hi_hbm, mb_hbm, qtm_hbm, kc_hbm, x2_hbm, y2_hbm, out_hbm,
                 hi_v, mb_v, qtm_v, x2_v, kc_v, y2_v, res_v):
    c = lax.axis_index("c")
    s_ = lax.axis_index("s")
    wid = s_ * 2 + c
    base = wid * _QPW

    pltpu.sync_copy(hi_hbm.at[pl.ds(base, _QPW)], hi_v.at[pl.ds(0, _QPW)])
    pltpu.sync_copy(mb_hbm.at[pl.ds(base, _QPW)], mb_v)
    pltpu.sync_copy(qtm_hbm.at[:, pl.ds(base, _QPW)],
                    qtm_v.at[:, pl.ds(0, _QPW)])
    pltpu.sync_copy(x2_hbm.at[pl.ds(base, _QPW)], x2_v.at[pl.ds(0, _QPW)])

    iota = lax.iota(jnp.int32, 16)
    iotaf = iota.astype(jnp.float32)
    bigf = jnp.full((16,), 16777216.0, jnp.float32)
    clipv = jnp.full((16,), 1e-12, jnp.float32)

    def qbody(i, carry):
        r0, r1 = carry
        hi_q = hi_v[pl.ds(i, 16)][0]
        bhi = jnp.full((16,), hi_q, jnp.float32)

        bmin = bigf
        for cch in range(_NBLK // 16):
            mbv = mb_v[i, pl.ds(cch * 16, 16)]
            mbc = jnp.maximum(mbv, clipv)
            cand = jnp.where(mbc <= bhi, cch * 16 + iotaf, bigf)
            bmin = jnp.minimum(bmin, cand)
        bstar = jnp.min(bmin).astype(jnp.int32)

        off = pl.multiple_of(bstar * _BK, _BK)
        pltpu.sync_copy(kc_hbm.at[:, pl.ds(off, _BK)], kc_v)
        pltpu.sync_copy(y2_hbm.at[bstar], y2_v)

        def bfr(x):
            u = lax.bitcast_convert_type(x, jnp.int32)
            r = (u + 0x7FFF + ((u >> 16) & 1)) & (-65536)
            return lax.bitcast_convert_type(r, jnp.float32)

        bqxm = bfr(jnp.full((16,), qtm_v[0, pl.ds(i, 16)][0], jnp.float32)) * -2.0
        bqym = bfr(jnp.full((16,), qtm_v[1, pl.ds(i, 16)][0], jnp.float32)) * -2.0
        bqzm = bfr(jnp.full((16,), qtm_v[2, pl.ds(i, 16)][0], jnp.float32)) * -2.0
        bx2 = jnp.full((16,), x2_v[pl.ds(i, 16)][0], jnp.float32)
        amask = bx2 == bx2

        rmin = bigf
        for j in range(_BK // 16):
            sl = pl.ds(j * 16, 16)
            kxv = kc_v[0, sl]
            kyv = kc_v[1, sl]
            kzv = kc_v[2, sl]
            y2 = y2_v[sl]
            def armor(x):
                return jnp.where(amask, x, bigf)

            p0 = bqxm * bfr(kxv)
            p1 = bqym * bfr(kyv)
            p2 = bqzm * bfr(kzv)
            s1 = armor(p0 + p1)
            bb1 = armor(s1 - p0)
            e1 = armor(p0 - (s1 - bb1)) + armor(p1 - bb1)
            s2 = armor(s1 + p2)
            bb2 = armor(s2 - s1)
            e2 = armor(s1 - (s2 - bb2)) + armor(p2 - bb2)
            t = s2 + armor(e1 + e2)
            d2 = (bx2 + y2) + t
            d2c = jnp.maximum(d2, clipv)
            cand = jnp.where(d2c <= bhi, j * 16 + iotaf, bigf)
            rmin = jnp.minimum(rmin, cand)
        fi = jnp.min(rmin).astype(jnp.int32)
        res = bstar * _BK + fi

        lane = jnp.where(iota == (i & 15), res, 0)
        grp = i >> 4
        r0 = jnp.where(jnp.full((16,), grp == 0), r0 | lane, r0)
        r1 = jnp.where(jnp.full((16,), grp == 1), r1 | lane, r1)
        return (r0, r1)

    z = jnp.zeros((16,), jnp.int32)
    r0, r1 = lax.fori_loop(0, _QPW, qbody, (z, z))
    res_v[pl.ds(0, 16)] = r0
    res_v[pl.ds(16, 16)] = r1
    pltpu.sync_copy(res_v, out_hbm.at[pl.ds(base, _QPW)])


def kernel(queries, keys, k):
    q, dim = queries.shape
    pdim = 8
    queries_p = jnp.pad(queries, ((0, 0), (0, pdim - dim)))
    keys_t = jnp.pad(keys.T, ((0, pdim - dim), (0, 0)))

    dist, hi, mb, y2a, x2o = pl.pallas_call(
        _phase1_body,
        grid=(_NBLK,),
        in_specs=[
            pl.BlockSpec((q, pdim), lambda i: (0, 0)),
            pl.BlockSpec((pdim, _BK), lambda i: (0, i)),
        ],
        out_specs=[
            pl.BlockSpec((q, 1), lambda i: (0, 0)),
            pl.BlockSpec((q, 1), lambda i: (0, 0)),
            pl.BlockSpec((1, q, 1), lambda i: (i, 0, 0)),
            pl.BlockSpec((1, 1, _BK), lambda i: (i, 0, 0)),
            pl.BlockSpec((q, 1), lambda i: (0, 0)),
        ],
        out_shape=[
            jax.ShapeDtypeStruct((q, 1), jnp.float32),
            jax.ShapeDtypeStruct((q, 1), jnp.float32),
            jax.ShapeDtypeStruct((_NBLK, q, 1), jnp.float32),
            jax.ShapeDtypeStruct((_NBLK, 1, _BK), jnp.float32),
            jax.ShapeDtypeStruct((q, 1), jnp.float32),
        ],
        scratch_shapes=[
            pltpu.VMEM((q, 1), jnp.float32),
        ],
    )(queries_p, keys_t)

    kcomb = keys.T
    qtm = queries.T

    mesh = plsc.VectorSubcoreMesh(core_axis_name="c", subcore_axis_name="s")
    phase2 = pl.kernel(
        _phase2_body,
        mesh=mesh,
        out_type=jax.ShapeDtypeStruct((q,), jnp.int32),
        compiler_params=pltpu.CompilerParams(needs_layout_passes=False),
        scratch_types=[
            pltpu.VMEM((_QPW + 16,), jnp.float32),
            pltpu.VMEM((_QPW, _NBLK), jnp.float32),
            pltpu.VMEM((3, _QPW + 16), jnp.float32),
            pltpu.VMEM((_QPW + 16,), jnp.float32),
            pltpu.VMEM((3, _BK), jnp.float32),
            pltpu.VMEM((_BK,), jnp.float32),
            pltpu.VMEM((_QPW,), jnp.int32),
        ],
    )
    mb_t = mb.reshape(_NBLK, q).T
    idxs = phase2(hi[:, 0], mb_t, qtm, kcomb,
                  x2o[:, 0], y2a.reshape(_NBLK, _BK))

    idx = idxs[:, None] + (jnp.asarray(k, dtype=idxs.dtype) - 1)
    return (dist, idx, idx[:, 0])
